# baseline (speedup 1.0000x reference)
import functools

import jax
import jax.numpy as jnp
from jax import lax
from jax.experimental import pallas as pl
from jax.experimental.pallas import tpu as pltpu
from jax.experimental.pallas import tpu_sc as plsc

D = 32
B = 16384
NC, NS, L = 2, 16, 16
NW = NC * NS
BPW = B // NW
CHUNK = 128
NCHUNK = BPW // CHUNK
GROUPS = BPW // L


def _distmult_body(h_hbm, r_hbm, t_hbm, ent_hbm, rel_hbm, out_hbm,
                   idx_h, idx_r, idx_t, h_rows, r_rows, t_rows, out_v, sem):
    wid = lax.axis_index("s") * NC + lax.axis_index("c")
    base = wid * BPW

    for j in range(NCHUNK):
        pltpu.sync_copy(h_hbm.at[pl.ds(base + j * CHUNK, CHUNK)], idx_h.at[j])
        pltpu.sync_copy(r_hbm.at[pl.ds(base + j * CHUNK, CHUNK)], idx_r.at[j])
        pltpu.sync_copy(t_hbm.at[pl.ds(base + j * CHUNK, CHUNK)], idx_t.at[j])

    copies = []
    for j in range(NCHUNK):
        dst = pl.ds(j * CHUNK, CHUNK)
        copies.append(pltpu.async_copy(ent_hbm.at[idx_h.at[j]], h_rows.at[dst], sem))
        copies.append(pltpu.async_copy(rel_hbm.at[idx_r.at[j]], r_rows.at[dst], sem))
        copies.append(pltpu.async_copy(ent_hbm.at[idx_t.at[j]], t_rows.at[dst], sem))
    for c in copies:
        c.wait()

    def group(g, carry):
        rows = g * L + lax.iota(jnp.int32, L)
        acc = jnp.zeros((L,), jnp.float32)
        for d in range(D):
            col = jnp.full((L,), d, jnp.int32)
            hv = plsc.load_gather(h_rows, [rows, col])
            rv = plsc.load_gather(r_rows, [rows, col])
            tv = plsc.load_gather(t_rows, [rows, col])
            acc = acc + hv * rv * tv
        out_v[pl.ds(g * L, L)] = acc
        return carry

    lax.fori_loop(0, GROUPS, group, 0)
    pltpu.sync_copy(out_v, out_hbm.at[pl.ds(base, BPW)])


@functools.partial(
    pl.kernel,
    out_type=jax.ShapeDtypeStruct((B,), jnp.float32),
    mesh=plsc.VectorSubcoreMesh(core_axis_name="c", subcore_axis_name="s"),
    scratch_types=[
        pltpu.VMEM((NCHUNK, CHUNK), jnp.int32),
        pltpu.VMEM((NCHUNK, CHUNK), jnp.int32),
        pltpu.VMEM((NCHUNK, CHUNK), jnp.int32),
        pltpu.VMEM((BPW, D), jnp.float32),
        pltpu.VMEM((BPW, D), jnp.float32),
        pltpu.VMEM((BPW, D), jnp.float32),
        pltpu.VMEM((BPW,), jnp.float32),
        pltpu.SemaphoreType.DMA,
    ],
    compiler_params=pltpu.CompilerParams(
        needs_layout_passes=False, use_tc_tiling_on_sc=False),
)
def _distmult_sc(h, r, t, ent, rel, out, *scratch):
    _distmult_body(h, r, t, ent, rel, out, *scratch)


def kernel(h, r, t, entity_embedding, relation_embedding):
    return _distmult_sc(h, r, t, entity_embedding, relation_embedding)


# speedup vs baseline: 3.1625x; 3.1625x over previous
import functools

import jax
import jax.numpy as jnp
from jax import lax
from jax.experimental import pallas as pl
from jax.experimental.pallas import tpu as pltpu
from jax.experimental.pallas import tpu_sc as plsc

D = 32
B = 16384
NW = 32
L = 16

NENT = 1000000
CHUNKW = 1024
NCHUNKS = (NENT + CHUNKW - 1) // CHUNKW
TAILW = 512
NTAIL = 64
ITEMCAP = 2048
BCAP = 256
MAXCH = 31
EROWS = 2 * B + L


def _k1_body(h_hbm, t_hbm, ent_t, tail_hbm, erows_hbm,
             idx_buf, items_id, items_pos, bucket_id, bucket_pos,
             chunk0, chunk1, tail_buf, stage, pstage, bcnt_s,
             sem_c0, sem_c1, sem_s, sem_i):
    wid = lax.axis_index("s") * 2 + lax.axis_index("c")
    start_ch = wid * 30 + jnp.minimum(wid, 17)
    n_ch = 30 + jnp.where(wid < 17, 1, 0)

    chunks = (chunk0, chunk1)
    sems = (sem_c0, sem_c1)

    def fire_dma(j, slot):
        g = start_ch + j
        is_tail = g == NCHUNKS - 1

        @pl.when(jnp.logical_not(is_tail))
        def _():
            pltpu.async_copy(ent_t.at[:, pl.ds(g * CHUNKW, CHUNKW)],
                             chunks[slot], sems[slot])

        @pl.when(is_tail)
        def _():
            pltpu.async_copy(
                ent_t.at[:, pl.ds((NCHUNKS - 1) * CHUNKW, TAILW)],
                chunks[slot].at[:, pl.ds(0, TAILW)], sems[slot])

    def wait_dma(j, slot):
        g = start_ch + j
        is_tail = g == NCHUNKS - 1

        @pl.when(jnp.logical_not(is_tail))
        def _():
            pltpu.make_async_copy(ent_t.at[:, pl.ds(0, CHUNKW)],
                                  chunks[slot], sems[slot]).wait()

        @pl.when(is_tail)
        def _():
            pltpu.make_async_copy(
                ent_t.at[:, pl.ds(0, TAILW)],
                chunks[slot].at[:, pl.ds(0, TAILW)], sems[slot]).wait()

    fire_dma(0, 0)
    fire_dma(1, 1)

    pltpu.async_copy(tail_hbm, tail_buf, sem_i).wait()

    lo = start_ch
    hi = start_ch + n_ch

    def filter_pass(which, n0):
        src = h_hbm if which == 0 else t_hbm
        pltpu.async_copy(src, idx_buf, sem_i).wait()

        def body(v, n):
            ids = idx_buf[pl.ds(v * L, L)]
            g = lax.shift_right_logical(ids, 10)
            m = jnp.logical_and(g >= lo, g < hi)
            posv = v * L + lax.iota(jnp.int32, L) + which * B
            plsc.store_compressed(items_id.at[pl.ds(n, L)], ids, mask=m)
            plsc.store_compressed(items_pos.at[pl.ds(n, L)], posv, mask=m)
            cnt = plsc.all_reduce_population_count(m)[0]
            return jnp.minimum(n + cnt, ITEMCAP)

        return lax.fori_loop(0, B // L, body, n0)

    n_items = filter_pass(1, filter_pass(0, jnp.int32(0)))

    for j in range(MAXCH):
        bcnt_s[j] = 0

    iota = lax.iota(jnp.int32, L)
    lane0 = iota == 0

    def bucket_body(i, carry):
        iid = items_id[pl.ds(i, L)][0]
        ipos = items_pos[pl.ds(i, L)][0]
        j = lax.shift_right_logical(iid, 10) - start_ch
        k = jnp.minimum(bcnt_s[j], BCAP - 1)
        slot = jnp.full((L,), j * BCAP + k, jnp.int32)
        plsc.store_scatter(bucket_id, [slot], jnp.full((L,), iid, jnp.int32),
                           mask=lane0)
        plsc.store_scatter(bucket_pos, [slot], jnp.full((L,), ipos, jnp.int32),
                           mask=lane0)
        bcnt_s[j] = k + 1
        return carry

    lax.fori_loop(0, n_items, bucket_body, 0)

    def scatter_batch(sb):
        @pl.when(sb == 0)
        def _():
            pltpu.async_copy(stage.at[0], erows_hbm.at[pstage.at[0]], sem_s)

        @pl.when(sb == 1)
        def _():
            pltpu.async_copy(stage.at[1], erows_hbm.at[pstage.at[1]], sem_s)

    def drain_one():
        pltpu.make_async_copy(stage.at[0], erows_hbm.at[pstage.at[0]],
                              sem_s).wait()

    def chunk_body(j, carry):
        kglob, pos_vec = carry
        slot = lax.rem(j, 2)

        @pl.when(slot == 0)
        def _():
            wait_dma(j, 0)

        @pl.when(slot == 1)
        def _():
            wait_dma(j, 1)

        m = bcnt_s[j]
        base_id = (start_ch + j) * CHUNKW

        def item_body(k, kc):
            kg, pvec = kc
            iid = bucket_id[pl.ds(j * BCAP + k, L)][0]
            ipos = bucket_pos[pl.ds(j * BCAP + k, L)][0]
            sslot = lax.rem(kg, 2 * L)
            sb = lax.div(sslot, L)
            sk = lax.rem(sslot, L)

            @pl.when(jnp.logical_and(sk == 0, kg >= 2 * L))
            def _():
                drain_one()

            col = jnp.full((L,), iid - base_id, jnp.int32)
            in_tail = iid >= NENT - NTAIL

            @pl.when(jnp.logical_and(slot == 0, jnp.logical_not(in_tail)))
            def _():
                v0 = plsc.load_gather(chunk0, [iota, col])
                v1 = plsc.load_gather(chunk0, [iota + L, col])
                stage[sb, sk, pl.ds(0, L)] = v0
                stage[sb, sk, pl.ds(L, L)] = v1

            @pl.when(jnp.logical_and(slot == 1, jnp.logical_not(in_tail)))
            def _():
                v0 = plsc.load_gather(chunk1, [iota, col])
                v1 = plsc.load_gather(chunk1, [iota + L, col])
                stage[sb, sk, pl.ds(0, L)] = v0
                stage[sb, sk, pl.ds(L, L)] = v1

            @pl.when(in_tail)
            def _():
                trow = jnp.full((L,), iid - (NENT - NTAIL), jnp.int32)
                v0 = plsc.load_gather(tail_buf, [trow, iota])
                v1 = plsc.load_gather(tail_buf, [trow, iota + L])
                stage[sb, sk, pl.ds(0, L)] = v0
                stage[sb, sk, pl.ds(L, L)] = v1

            pvec = jnp.where(iota == sk, ipos, pvec)

            @pl.when(sk == L - 1)
            def _():
                @pl.when(sb == 0)
                def _():
                    pstage[0, pl.ds(0, L)] = pvec

                @pl.when(sb == 1)
                def _():
                    pstage[1, pl.ds(0, L)] = pvec

                scatter_batch(sb)

            return kg + 1, pvec

        kglob, pos_vec = lax.fori_loop(0, m, item_body, (kglob, pos_vec))

        @pl.when(j + 2 < n_ch)
        def _():
            @pl.when(slot == 0)
            def _():
                fire_dma(j + 2, 0)

            @pl.when(slot == 1)
            def _():
                fire_dma(j + 2, 1)

        return kglob, pos_vec

    kglob, pos_vec = lax.fori_loop(
        0, n_ch, chunk_body, (jnp.int32(0), jnp.zeros((L,), jnp.int32)))

    rem = lax.rem(kglob, L)
    sb_last = lax.rem(lax.div(kglob, L), 2)

    @pl.when(rem > 0)
    def _():
        pvec = jnp.where(iota >= rem, 2 * B + iota, pos_vec)

        @pl.when(sb_last == 0)
        def _():
            pstage[0, pl.ds(0, L)] = pvec

        @pl.when(sb_last == 1)
        def _():
            pstage[1, pl.ds(0, L)] = pvec

        scatter_batch(sb_last)

    nscat = lax.div(kglob + L - 1, L)

    def drain_body(b, c):
        @pl.when(b < jnp.minimum(nscat, 2))
        def _():
            drain_one()

        return c

    lax.fori_loop(0, 2, drain_body, 0)


def _k2_body(erows_hbm, r_hbm, rel_t, out_hbm,
             hbuf, tbuf, relbuf, ridx, out_v, sem):
    wid = lax.axis_index("s") * 2 + lax.axis_index("c")
    base = wid * (B // NW)

    pltpu.async_copy(rel_t, relbuf, sem).wait()
    pltpu.async_copy(r_hbm.at[pl.ds(base, B // NW)], ridx, sem).wait()

    for p in range(2):
        rb = base + p * 256
        pltpu.async_copy(erows_hbm.at[pl.ds(rb, 256)], hbuf, sem).wait()
        pltpu.async_copy(erows_hbm.at[pl.ds(B + rb, 256)], tbuf, sem).wait()

        def group(g, carry):
            rows = g * L + lax.iota(jnp.int32, L)
            rvals = ridx[pl.ds(p * 256 + g * L, L)]
            acc = jnp.zeros((L,), jnp.float32)
            for d in range(D):
                cold = jnp.full((L,), d, jnp.int32)
                hv = plsc.load_gather(hbuf, [rows, cold])
                tv = plsc.load_gather(tbuf, [rows, cold])
                rv = plsc.load_gather(relbuf, [cold, rvals])
                acc = acc + hv * tv * rv
            out_v[pl.ds(p * 256 + g * L, L)] = acc
            return carry

        lax.fori_loop(0, 256 // L, group, 0)

    pltpu.sync_copy(out_v, out_hbm.at[pl.ds(base, B // NW)])


_MESH = plsc.VectorSubcoreMesh(core_axis_name="c", subcore_axis_name="s")


@functools.partial(
    pl.kernel,
    out_type=jax.ShapeDtypeStruct((EROWS, 128), jnp.float32),
    mesh=_MESH,
    scratch_types=[
        pltpu.VMEM((B,), jnp.int32),
        pltpu.VMEM((ITEMCAP + L,), jnp.int32),
        pltpu.VMEM((ITEMCAP + L,), jnp.int32),
        pltpu.VMEM((MAXCH * BCAP + L,), jnp.int32),
        pltpu.VMEM((MAXCH * BCAP + L,), jnp.int32),
        pltpu.VMEM((D, CHUNKW), jnp.float32),
        pltpu.VMEM((D, CHUNKW), jnp.float32),
        pltpu.VMEM((NTAIL, D), jnp.float32),
        pltpu.VMEM((2, L, 128), jnp.float32),
        pltpu.VMEM((2, L), jnp.int32),
        pltpu.SMEM((MAXCH,), jnp.int32),
        pltpu.SemaphoreType.DMA,
        pltpu.SemaphoreType.DMA,
        pltpu.SemaphoreType.DMA,
        pltpu.SemaphoreType.DMA,
    ],
    compiler_params=pltpu.CompilerParams(needs_layout_passes=False),
)
def _k1(h, t, ent_t, tail, erows, *scratch):
    _k1_body(h, t, ent_t, tail, erows, *scratch)


@functools.partial(
    pl.kernel,
    out_type=jax.ShapeDtypeStruct((B,), jnp.float32),
    mesh=_MESH,
    scratch_types=[
        pltpu.VMEM((256, 128), jnp.float32),
        pltpu.VMEM((256, 128), jnp.float32),
        pltpu.VMEM((D, 1000), jnp.float32),
        pltpu.VMEM((B // NW,), jnp.int32),
        pltpu.VMEM((B // NW,), jnp.float32),
        pltpu.SemaphoreType.DMA,
    ],
    compiler_params=pltpu.CompilerParams(needs_layout_passes=False),
)
def _k2(erows, r, rel_t, out, *scratch):
    _k2_body(erows, r, rel_t, out, *scratch)


def kernel(h, r, t, entity_embedding, relation_embedding):
    tail = entity_embedding[NENT - NTAIL:]
    erows = _k1(h, t, entity_embedding.T, tail)
    return _k2(erows, r, relation_embedding.T)


# speedup vs baseline: 3.1858x; 1.0074x over previous
import functools

import jax
import jax.numpy as jnp
from jax import lax
from jax.experimental import pallas as pl
from jax.experimental.pallas import tpu as pltpu
from jax.experimental.pallas import tpu_sc as plsc

D = 32
B = 16384
NW = 32
L = 16

NENT = 1000000
CHUNKW = 1024
NCHUNKS = (NENT + CHUNKW - 1) // CHUNKW
TAILW = 512
NTAIL = 64
ITEMCAP = 2048
BCAP = 256
MAXCH = 31
EROWS = 2 * B + L


def _k1_body(h_hbm, t_hbm, ent_t, tail_hbm, erows_hbm,
             idx_buf, items_id, items_pos, bucket_id, bucket_pos,
             chunk0, chunk1, tail_buf, stage, pstage, bcnt_s,
             sem_c0, sem_c1, sem_s, sem_i):
    wid = lax.axis_index("s") * 2 + lax.axis_index("c")
    start_ch = wid * 30 + jnp.minimum(wid, 17)
    n_ch = 30 + jnp.where(wid < 17, 1, 0)

    chunks = (chunk0, chunk1)
    sems = (sem_c0, sem_c1)

    def fire_dma(j, slot):
        g = start_ch + j
        is_tail = g == NCHUNKS - 1

        @pl.when(jnp.logical_not(is_tail))
        def _():
            pltpu.async_copy(ent_t.at[:, pl.ds(g * CHUNKW, CHUNKW)],
                             chunks[slot], sems[slot])

        @pl.when(is_tail)
        def _():
            pltpu.async_copy(
                ent_t.at[:, pl.ds((NCHUNKS - 1) * CHUNKW, TAILW)],
                chunks[slot].at[:, pl.ds(0, TAILW)], sems[slot])

    def wait_dma(j, slot):
        g = start_ch + j
        is_tail = g == NCHUNKS - 1

        @pl.when(jnp.logical_not(is_tail))
        def _():
            pltpu.make_async_copy(ent_t.at[:, pl.ds(0, CHUNKW)],
                                  chunks[slot], sems[slot]).wait()

        @pl.when(is_tail)
        def _():
            pltpu.make_async_copy(
                ent_t.at[:, pl.ds(0, TAILW)],
                chunks[slot].at[:, pl.ds(0, TAILW)], sems[slot]).wait()

    fire_dma(0, 0)
    fire_dma(1, 1)

    pltpu.async_copy(tail_hbm, tail_buf, sem_i).wait()

    lo = start_ch
    hi = start_ch + n_ch

    def filter_pass(which, n0):
        src = h_hbm if which == 0 else t_hbm
        pltpu.async_copy(src, idx_buf, sem_i).wait()

        def body(v, n):
            ids = idx_buf[pl.ds(v * L, L)]
            g = lax.shift_right_logical(ids, 10)
            m = jnp.logical_and(g >= lo, g < hi)
            posv = v * L + lax.iota(jnp.int32, L) + which * B
            plsc.store_compressed(items_id.at[pl.ds(n, L)], ids, mask=m)
            plsc.store_compressed(items_pos.at[pl.ds(n, L)], posv, mask=m)
            cnt = plsc.all_reduce_population_count(m)[0]
            return jnp.minimum(n + cnt, ITEMCAP)

        return lax.fori_loop(0, B // L, body, n0, unroll=8)

    n_items = filter_pass(1, filter_pass(0, jnp.int32(0)))

    for j in range(MAXCH):
        bcnt_s[j] = 0

    iota = lax.iota(jnp.int32, L)
    lane0 = iota == 0

    def bucket_body(i, carry):
        iid = items_id[pl.ds(i, L)][0]
        ipos = items_pos[pl.ds(i, L)][0]
        j = lax.shift_right_logical(iid, 10) - start_ch
        k = jnp.minimum(bcnt_s[j], BCAP - 1)
        slot = jnp.full((L,), j * BCAP + k, jnp.int32)
        plsc.store_scatter(bucket_id, [slot], jnp.full((L,), iid, jnp.int32),
                           mask=lane0)
        plsc.store_scatter(bucket_pos, [slot], jnp.full((L,), ipos, jnp.int32),
                           mask=lane0)
        bcnt_s[j] = k + 1
        return carry

    lax.fori_loop(0, n_items, bucket_body, 0)

    def scatter_batch(sb):
        @pl.when(sb == 0)
        def _():
            pltpu.async_copy(stage.at[0], erows_hbm.at[pstage.at[0]], sem_s)

        @pl.when(sb == 1)
        def _():
            pltpu.async_copy(stage.at[1], erows_hbm.at[pstage.at[1]], sem_s)

    def drain_one():
        pltpu.make_async_copy(stage.at[0], erows_hbm.at[pstage.at[0]],
                              sem_s).wait()

    def chunk_body(j, carry):
        kglob, pos_vec = carry
        slot = lax.rem(j, 2)

        @pl.when(slot == 0)
        def _():
            wait_dma(j, 0)

        @pl.when(slot == 1)
        def _():
            wait_dma(j, 1)

        m = bcnt_s[j]
        base_id = (start_ch + j) * CHUNKW

        def item_body(k, kc):
            kg, pvec = kc
            iid = bucket_id[pl.ds(j * BCAP + k, L)][0]
            ipos = bucket_pos[pl.ds(j * BCAP + k, L)][0]
            sslot = lax.rem(kg, 2 * L)
            sb = lax.div(sslot, L)
            sk = lax.rem(sslot, L)

            @pl.when(jnp.logical_and(sk == 0, kg >= 2 * L))
            def _():
                drain_one()

            col = jnp.full((L,), iid - base_id, jnp.int32)
            in_tail = iid >= NENT - NTAIL

            @pl.when(jnp.logical_and(slot == 0, jnp.logical_not(in_tail)))
            def _():
                v0 = plsc.load_gather(chunk0, [iota, col])
                v1 = plsc.load_gather(chunk0, [iota + L, col])
                stage[sb, sk, pl.ds(0, L)] = v0
                stage[sb, sk, pl.ds(L, L)] = v1

            @pl.when(jnp.logical_and(slot == 1, jnp.logical_not(in_tail)))
            def _():
                v0 = plsc.load_gather(chunk1, [iota, col])
                v1 = plsc.load_gather(chunk1, [iota + L, col])
                stage[sb, sk, pl.ds(0, L)] = v0
                stage[sb, sk, pl.ds(L, L)] = v1

            @pl.when(in_tail)
            def _():
                trow = jnp.full((L,), iid - (NENT - NTAIL), jnp.int32)
                v0 = plsc.load_gather(tail_buf, [trow, iota])
                v1 = plsc.load_gather(tail_buf, [trow, iota + L])
                stage[sb, sk, pl.ds(0, L)] = v0
                stage[sb, sk, pl.ds(L, L)] = v1

            pvec = jnp.where(iota == sk, ipos, pvec)

            @pl.when(sk == L - 1)
            def _():
                @pl.when(sb == 0)
                def _():
                    pstage[0, pl.ds(0, L)] = pvec

                @pl.when(sb == 1)
                def _():
                    pstage[1, pl.ds(0, L)] = pvec

                scatter_batch(sb)

            return kg + 1, pvec

        kglob, pos_vec = lax.fori_loop(0, m, item_body, (kglob, pos_vec))

        @pl.when(j + 2 < n_ch)
        def _():
            @pl.when(slot == 0)
            def _():
                fire_dma(j + 2, 0)

            @pl.when(slot == 1)
            def _():
                fire_dma(j + 2, 1)

        return kglob, pos_vec

    kglob, pos_vec = lax.fori_loop(
        0, n_ch, chunk_body, (jnp.int32(0), jnp.zeros((L,), jnp.int32)))

    rem = lax.rem(kglob, L)
    sb_last = lax.rem(lax.div(kglob, L), 2)

    @pl.when(rem > 0)
    def _():
        pvec = jnp.where(iota >= rem, 2 * B + iota, pos_vec)

        @pl.when(sb_last == 0)
        def _():
            pstage[0, pl.ds(0, L)] = pvec

        @pl.when(sb_last == 1)
        def _():
            pstage[1, pl.ds(0, L)] = pvec

        scatter_batch(sb_last)

    nscat = lax.div(kglob + L - 1, L)

    def drain_body(b, c):
        @pl.when(b < jnp.minimum(nscat, 2))
        def _():
            drain_one()

        return c

    lax.fori_loop(0, 2, drain_body, 0)


def _k2_body(erows_hbm, r_hbm, rel_t, out_hbm,
             hbuf0, tbuf0, hbuf1, tbuf1, relbuf, ridx, out_v,
             sem_r, sem_h0, sem_t0, sem_h1, sem_t1):
    wid = lax.axis_index("s") * 2 + lax.axis_index("c")
    base = wid * (B // NW)
    PB = 128
    hbufs = (hbuf0, hbuf1)
    tbufs = (tbuf0, tbuf1)
    hsems = (sem_h0, sem_h1)
    tsems = (sem_t0, sem_t1)

    def fetch(p):
        s = p % 2
        rb = base + p * PB
        pltpu.async_copy(erows_hbm.at[pl.ds(rb, PB)], hbufs[s], hsems[s])
        pltpu.async_copy(erows_hbm.at[pl.ds(B + rb, PB)], tbufs[s], tsems[s])

    fetch(0)
    pltpu.async_copy(rel_t, relbuf, sem_r)
    pltpu.async_copy(r_hbm.at[pl.ds(base, B // NW)], ridx, sem_r)
    fetch(1)
    pltpu.make_async_copy(rel_t, relbuf, sem_r).wait()
    pltpu.make_async_copy(r_hbm.at[pl.ds(base, B // NW)], ridx, sem_r).wait()

    for p in range(4):
        s = p % 2
        hbuf, tbuf = hbufs[s], tbufs[s]
        rb = base + p * PB
        pltpu.make_async_copy(erows_hbm.at[pl.ds(rb, PB)], hbuf,
                              hsems[s]).wait()
        pltpu.make_async_copy(erows_hbm.at[pl.ds(B + rb, PB)], tbuf,
                              tsems[s]).wait()

        def group(g, carry):
            rows = g * L + lax.iota(jnp.int32, L)
            rvals = ridx[pl.ds(p * PB + g * L, L)]
            acc = jnp.zeros((L,), jnp.float32)
            for d in range(D):
                cold = jnp.full((L,), d, jnp.int32)
                hv = plsc.load_gather(hbuf, [rows, cold])
                tv = plsc.load_gather(tbuf, [rows, cold])
                rv = plsc.load_gather(relbuf, [cold, rvals])
                acc = acc + hv * tv * rv
            out_v[pl.ds(p * PB + g * L, L)] = acc
            return carry

        lax.fori_loop(0, PB // L, group, 0)

        if p + 2 < 4:
            fetch(p + 2)

    pltpu.sync_copy(out_v, out_hbm.at[pl.ds(base, B // NW)])


_MESH = plsc.VectorSubcoreMesh(core_axis_name="c", subcore_axis_name="s")


@functools.partial(
    pl.kernel,
    out_type=jax.ShapeDtypeStruct((EROWS, 128), jnp.float32),
    mesh=_MESH,
    scratch_types=[
        pltpu.VMEM((B,), jnp.int32),
        pltpu.VMEM((ITEMCAP + L,), jnp.int32),
        pltpu.VMEM((ITEMCAP + L,), jnp.int32),
        pltpu.VMEM((MAXCH * BCAP + L,), jnp.int32),
        pltpu.VMEM((MAXCH * BCAP + L,), jnp.int32),
        pltpu.VMEM((D, CHUNKW), jnp.float32),
        pltpu.VMEM((D, CHUNKW), jnp.float32),
        pltpu.VMEM((NTAIL, D), jnp.float32),
        pltpu.VMEM((2, L, 128), jnp.float32),
        pltpu.VMEM((2, L), jnp.int32),
        pltpu.SMEM((MAXCH,), jnp.int32),
        pltpu.SemaphoreType.DMA,
        pltpu.SemaphoreType.DMA,
        pltpu.SemaphoreType.DMA,
        pltpu.SemaphoreType.DMA,
    ],
    compiler_params=pltpu.CompilerParams(needs_layout_passes=False),
)
def _k1(h, t, ent_t, tail, erows, *scratch):
    _k1_body(h, t, ent_t, tail, erows, *scratch)


@functools.partial(
    pl.kernel,
    out_type=jax.ShapeDtypeStruct((B,), jnp.float32),
    mesh=_MESH,
    scratch_types=[
        pltpu.VMEM((128, 128), jnp.float32),
        pltpu.VMEM((128, 128), jnp.float32),
        pltpu.VMEM((128, 128), jnp.float32),
        pltpu.VMEM((128, 128), jnp.float32),
        pltpu.VMEM((D, 1000), jnp.float32),
        pltpu.VMEM((B // NW,), jnp.int32),
        pltpu.VMEM((B // NW,), jnp.float32),
        pltpu.SemaphoreType.DMA,
        pltpu.SemaphoreType.DMA,
        pltpu.SemaphoreType.DMA,
        pltpu.SemaphoreType.DMA,
        pltpu.SemaphoreType.DMA,
    ],
    compiler_params=pltpu.CompilerParams(needs_layout_passes=False),
)
def _k2(erows, r, rel_t, out, *scratch):
    _k2_body(erows, r, rel_t, out, *scratch)


def kernel(h, r, t, entity_embedding, relation_embedding):
    tail = entity_embedding[NENT - NTAIL:]
    erows = _k1(h, t, entity_embedding.T, tail)
    return _k2(erows, r, relation_embedding.T)
